# baseline (speedup 1.0000x reference)
import functools

import jax
import jax.numpy as jnp
from jax import lax
from jax.experimental import pallas as pl
from jax.experimental.pallas import tpu as pltpu
from jax.experimental.pallas import tpu_sc as plsc

_B = 16384
_SPARSE_START = 13
_FIELD_NUM = 26
_VOCAB = 100000
_DIM = 16
_XCOLS = _SPARSE_START + _FIELD_NUM

_NC = 2
_NS = 16
_NW = _NC * _NS
_LANES = 16

_ROWS_PER_W = _B // _NW
_CH = 128
_NITER = _ROWS_PER_W // _CH

_OUT_COLS = _SPARSE_START + _FIELD_NUM * _DIM

_VBLK = 512
_NFULL = _VOCAB // _VBLK
_VMAIN = _NFULL * _VBLK
_TAIL1 = 128
_TAIL2 = _VOCAB - _VMAIN - _TAIL1
_WPAD = 12504
_VPAD = _WPAD * 8
_NBLK = _FIELD_NUM * _NFULL


def _sc_relayout(tab_t, tails):
    mesh = plsc.VectorSubcoreMesh(core_axis_name="c", subcore_axis_name="s")

    @functools.partial(
        pl.kernel,
        mesh=mesh,
        compiler_params=pltpu.CompilerParams(
            use_tc_tiling_on_sc=True, needs_layout_passes=False
        ),
        out_type=jax.ShapeDtypeStruct((_FIELD_NUM, _WPAD, 128), jnp.float32),
        scratch_types=[
            pltpu.VMEM((_DIM, _VBLK), jnp.float32),
            pltpu.VMEM((_VBLK // 8, 128), jnp.float32),
            pltpu.VMEM((_TAIL2, _DIM), jnp.float32),
        ],
    )
    def k(tab_hbm, tails_hbm, out_hbm, stg, cmp, stg2):
        wid = lax.axis_index("s") * _NC + lax.axis_index("c")
        lane = lax.iota(jnp.int32, _LANES)
        niter = (_NBLK + _NW - 1) // _NW

        def transpose_into(n_wide, src, dst):
            def body(w, c):
                for kk in range(8):
                    col = jnp.full((_LANES,), kk, jnp.int32) + w * 8
                    vals = plsc.load_gather(src, [lane, col])
                    dst[w, pl.ds(kk * _DIM, _DIM)] = vals
                return c

            lax.fori_loop(0, n_wide, body, 0)

        def step(it, carry):
            bid = it * _NW + wid

            @pl.when(bid < _NBLK)
            def _():
                f = bid // _NFULL
                vb = pl.multiple_of((bid % _NFULL) * _VBLK, _VBLK)
                pltpu.sync_copy(tab_hbm.at[f, :, pl.ds(vb, _VBLK)], stg)
                transpose_into(_VBLK // 8, stg, cmp)
                wb = pl.multiple_of((bid % _NFULL) * (_VBLK // 8), _VBLK // 8)
                pltpu.sync_copy(cmp, out_hbm.at[f, pl.ds(wb, _VBLK // 8)])

            return carry

        lax.fori_loop(0, niter, step, 0)

        @pl.when(wid < _FIELD_NUM)
        def _():
            f = wid
            pltpu.sync_copy(
                tab_hbm.at[f, :, pl.ds(pl.multiple_of(_VMAIN, 128), _TAIL1)],
                stg.at[:, pl.ds(0, _TAIL1)],
            )
            transpose_into(_TAIL1 // 8, stg, cmp)
            pltpu.sync_copy(tails_hbm.at[f], stg2)
            for w in range(_TAIL2 // 8):
                for kk in range(8):
                    cmp[_TAIL1 // 8 + w, pl.ds(kk * _DIM, _DIM)] = (
                        stg2[w * 8 + kk, pl.ds(0, _DIM)]
                    )
            pltpu.sync_copy(
                cmp.at[pl.ds(0, 24)],
                out_hbm.at[f, pl.ds(pl.multiple_of(_VMAIN // 8, 8), 24)],
            )

    return k(tab_t, tails)


def _sc_embed(x, table_flat):
    mesh = plsc.VectorSubcoreMesh(core_axis_name="c", subcore_axis_name="s")

    @functools.partial(
        pl.kernel,
        mesh=mesh,
        compiler_params=pltpu.CompilerParams(
            use_tc_tiling_on_sc=False, needs_layout_passes=False
        ),
        out_type=jax.ShapeDtypeStruct((_B, _OUT_COLS), jnp.float32),
        scratch_types=[
            pltpu.VMEM((_CH, _XCOLS), jnp.int32),
            pltpu.VMEM((_FIELD_NUM, _CH), jnp.int32),
            pltpu.VMEM((_FIELD_NUM, _CH, _DIM), jnp.float32),
            pltpu.VMEM((_CH, _OUT_COLS), jnp.float32),
            pltpu.SemaphoreType.DMA,
        ],
    )
    def k(x_hbm, tab_hbm, out_hbm, xs_v, idx_v, emb_v, row_v, gsem):
        wid = lax.axis_index("s") * _NC + lax.axis_index("c")
        w_base = wid * _ROWS_PER_W
        row_iota = lax.iota(jnp.int32, _LANES)

        def chunk(it, carry):
            base = pl.multiple_of(w_base + it * _CH, _CH)

            pltpu.sync_copy(x_hbm.at[pl.ds(base, _CH)], xs_v)

            for f in range(_FIELD_NUM):
                bias = f * _VPAD
                col = jnp.full((_LANES,), _SPARSE_START + f, jnp.int32)
                for j in range(_CH // _LANES):
                    rows = row_iota + (j * _LANES)
                    vals = plsc.load_gather(xs_v, [rows, col])
                    idx_v[f, pl.ds(j * _LANES, _LANES)] = vals + bias

            handles = []
            for f in range(_FIELD_NUM):
                handles.append(
                    pltpu.async_copy(tab_hbm.at[idx_v.at[f]], emb_v.at[f], gsem)
                )
            for h in handles:
                h.wait()

            def put_row(r, c):
                d = xs_v[r, pl.ds(0, _LANES)].astype(jnp.float32)
                row_v[r, pl.ds(0, _LANES)] = d
                for f in range(_FIELD_NUM):
                    row_v[r, pl.ds(_SPARSE_START + f * _DIM, _DIM)] = (
                        emb_v[f, r, pl.ds(0, _DIM)]
                    )
                return c

            lax.fori_loop(0, _CH, put_row, 0)

            pltpu.sync_copy(row_v, out_hbm.at[pl.ds(base, _CH)])
            return carry

        lax.fori_loop(0, _NITER, chunk, 0)

    return k(x, table_flat)


def kernel(X, tables):
    tab_t = jnp.transpose(tables, (0, 2, 1))
    tails = tables[:, _VMAIN + _TAIL1:, :]
    wide = _sc_relayout(tab_t, tails)
    table_flat = wide.reshape(_FIELD_NUM * _VPAD, _DIM)
    return _sc_embed(X, table_flat)


# speedup vs baseline: 1.0719x; 1.0719x over previous
import functools

import jax
import jax.numpy as jnp
from jax import lax
from jax.experimental import pallas as pl
from jax.experimental.pallas import tpu as pltpu
from jax.experimental.pallas import tpu_sc as plsc

_B = 16384
_SPARSE_START = 13
_FIELD_NUM = 26
_VOCAB = 100000
_DIM = 16
_XCOLS = _SPARSE_START + _FIELD_NUM

_NC = 2
_NS = 16
_NW = _NC * _NS
_LANES = 16

_ROWS_PER_W = _B // _NW
_CH = 128
_NITER = _ROWS_PER_W // _CH

_OUT_COLS = _SPARSE_START + _FIELD_NUM * _DIM

_VBLK = 2048
_NFULL = _VOCAB // _VBLK
_VMAIN = _NFULL * _VBLK
_TAIL1 = 1664
_TAIL2 = _VOCAB - _VMAIN - _TAIL1
_WPAD = 12504
_VPAD = _WPAD * 8
_NBLK = _FIELD_NUM * _NFULL


def _sc_relayout(tab_t, tails):
    mesh = plsc.VectorSubcoreMesh(core_axis_name="c", subcore_axis_name="s")

    @functools.partial(
        pl.kernel,
        mesh=mesh,
        compiler_params=pltpu.CompilerParams(
            use_tc_tiling_on_sc=True, needs_layout_passes=False
        ),
        out_type=jax.ShapeDtypeStruct((_FIELD_NUM, _WPAD, 128), jnp.float32),
        scratch_types=[
            pltpu.VMEM((_DIM, _VBLK), jnp.float32),
            pltpu.VMEM((_VBLK // 8, 128), jnp.float32),
            pltpu.VMEM((_TAIL2, _DIM), jnp.float32),
        ],
    )
    def k(tab_hbm, tails_hbm, out_hbm, stg, cmp, stg2):
        wid = lax.axis_index("s") * _NC + lax.axis_index("c")
        lane = lax.iota(jnp.int32, _LANES)
        niter = (_NBLK + _NW - 1) // _NW

        def transpose_into(n_wide, src, dst):
            def body(w, c):
                for kk in range(8):
                    col = jnp.full((_LANES,), kk, jnp.int32) + w * 8
                    vals = plsc.load_gather(src, [lane, col])
                    dst[w, pl.ds(kk * _DIM, _DIM)] = vals
                return c

            lax.fori_loop(0, n_wide, body, 0)

        def step(it, carry):
            bid = it * _NW + wid

            @pl.when(bid < _NBLK)
            def _():
                f = bid // _NFULL
                vb = pl.multiple_of((bid % _NFULL) * _VBLK, _VBLK)
                pltpu.sync_copy(tab_hbm.at[f, :, pl.ds(vb, _VBLK)], stg)
                transpose_into(_VBLK // 8, stg, cmp)
                wb = pl.multiple_of((bid % _NFULL) * (_VBLK // 8), _VBLK // 8)
                pltpu.sync_copy(cmp, out_hbm.at[f, pl.ds(wb, _VBLK // 8)])

            return carry

        lax.fori_loop(0, niter, step, 0)

        @pl.when(wid < _FIELD_NUM)
        def _():
            f = wid
            pltpu.sync_copy(
                tab_hbm.at[f, :, pl.ds(pl.multiple_of(_VMAIN, 128), _TAIL1)],
                stg.at[:, pl.ds(0, _TAIL1)],
            )
            transpose_into(_TAIL1 // 8, stg, cmp)
            pltpu.sync_copy(tails_hbm.at[f], stg2)
            for w in range(_TAIL2 // 8):
                for kk in range(8):
                    cmp[_TAIL1 // 8 + w, pl.ds(kk * _DIM, _DIM)] = (
                        stg2[w * 8 + kk, pl.ds(0, _DIM)]
                    )
            n_tail = (_TAIL1 + _TAIL2) // 8 + 4
            pltpu.sync_copy(
                cmp.at[pl.ds(0, n_tail)],
                out_hbm.at[f, pl.ds(pl.multiple_of(_VMAIN // 8, 8), n_tail)],
            )

    return k(tab_t, tails)


def _sc_embed(x, table_flat):
    mesh = plsc.VectorSubcoreMesh(core_axis_name="c", subcore_axis_name="s")

    @functools.partial(
        pl.kernel,
        mesh=mesh,
        compiler_params=pltpu.CompilerParams(
            use_tc_tiling_on_sc=False, needs_layout_passes=False
        ),
        out_type=jax.ShapeDtypeStruct((_B, _OUT_COLS), jnp.float32),
        scratch_types=[
            pltpu.VMEM((_CH, _XCOLS), jnp.int32),
            pltpu.VMEM((_FIELD_NUM, _CH), jnp.int32),
            pltpu.VMEM((_FIELD_NUM, _CH, _DIM), jnp.float32),
            pltpu.VMEM((_CH, _OUT_COLS), jnp.float32),
            pltpu.SemaphoreType.DMA,
        ],
    )
    def k(x_hbm, tab_hbm, out_hbm, xs_v, idx_v, emb_v, row_v, gsem):
        wid = lax.axis_index("s") * _NC + lax.axis_index("c")
        w_base = wid * _ROWS_PER_W
        row_iota = lax.iota(jnp.int32, _LANES)

        def chunk(it, carry):
            base = pl.multiple_of(w_base + it * _CH, _CH)

            pltpu.sync_copy(x_hbm.at[pl.ds(base, _CH)], xs_v)

            for f in range(_FIELD_NUM):
                bias = f * _VPAD
                col = jnp.full((_LANES,), _SPARSE_START + f, jnp.int32)
                for j in range(_CH // _LANES):
                    rows = row_iota + (j * _LANES)
                    vals = plsc.load_gather(xs_v, [rows, col])
                    idx_v[f, pl.ds(j * _LANES, _LANES)] = vals + bias

            handles = []
            for f in range(_FIELD_NUM):
                handles.append(
                    pltpu.async_copy(tab_hbm.at[idx_v.at[f]], emb_v.at[f], gsem)
                )
            for h in handles:
                h.wait()

            def put_row(r, c):
                d = xs_v[r, pl.ds(0, _LANES)].astype(jnp.float32)
                row_v[r, pl.ds(0, _LANES)] = d
                for f in range(_FIELD_NUM):
                    row_v[r, pl.ds(_SPARSE_START + f * _DIM, _DIM)] = (
                        emb_v[f, r, pl.ds(0, _DIM)]
                    )
                return c

            lax.fori_loop(0, _CH, put_row, 0)

            pltpu.sync_copy(row_v, out_hbm.at[pl.ds(base, _CH)])
            return carry

        lax.fori_loop(0, _NITER, chunk, 0)

    return k(x, table_flat)


def kernel(X, tables):
    tab_t = jnp.transpose(tables, (0, 2, 1))
    tails = tables[:, _VMAIN + _TAIL1:, :]
    wide = _sc_relayout(tab_t, tails)
    table_flat = wide.reshape(_FIELD_NUM * _VPAD, _DIM)
    return _sc_embed(X, table_flat)


# speedup vs baseline: 1.1528x; 1.0754x over previous
import functools

import jax
import jax.numpy as jnp
from jax import lax
from jax.experimental import pallas as pl
from jax.experimental.pallas import tpu as pltpu
from jax.experimental.pallas import tpu_sc as plsc

_B = 16384
_SPARSE_START = 13
_FIELD_NUM = 26
_VOCAB = 100000
_DIM = 16
_XCOLS = _SPARSE_START + _FIELD_NUM

_NC = 2
_NS = 16
_NW = _NC * _NS
_LANES = 16

_ROWS_PER_W = _B // _NW
_CH = 128
_NITER = _ROWS_PER_W // _CH

_OUT_COLS = _SPARSE_START + _FIELD_NUM * _DIM


def _sc_embed(x, tables):
    mesh = plsc.VectorSubcoreMesh(core_axis_name="c", subcore_axis_name="s")

    @functools.partial(
        pl.kernel,
        mesh=mesh,
        compiler_params=pltpu.CompilerParams(
            use_tc_tiling_on_sc=False, needs_layout_passes=False
        ),
        out_type=jax.ShapeDtypeStruct((_B, _OUT_COLS), jnp.float32),
        scratch_types=[
            pltpu.VMEM((_CH, _XCOLS), jnp.int32),
            pltpu.VMEM((_FIELD_NUM, _CH), jnp.int32),
            pltpu.VMEM((_FIELD_NUM, _CH, _DIM), jnp.float32),
            pltpu.VMEM((_CH, _OUT_COLS), jnp.float32),
            pltpu.SemaphoreType.DMA,
        ],
    )
    def k(x_hbm, tab_hbm, out_hbm, xs_v, idx_v, emb_v, row_v, gsem):
        wid = lax.axis_index("s") * _NC + lax.axis_index("c")
        w_base = wid * _ROWS_PER_W

        def chunk(it, carry):
            base = pl.multiple_of(w_base + it * _CH, _CH)

            pltpu.sync_copy(x_hbm.at[pl.ds(base, _CH)], xs_v)

            row_iota = lax.iota(jnp.int32, _LANES)
            for f in range(_FIELD_NUM):
                col = jnp.full((_LANES,), _SPARSE_START + f, jnp.int32)
                for j in range(_CH // _LANES):
                    rows = row_iota + (j * _LANES)
                    vals = plsc.load_gather(xs_v, [rows, col])
                    idx_v[f, pl.ds(j * _LANES, _LANES)] = vals

            handles = []
            for f in range(_FIELD_NUM):
                handles.append(
                    pltpu.async_copy(
                        tab_hbm.at[f].at[idx_v.at[f]], emb_v.at[f], gsem
                    )
                )
            for h in handles:
                h.wait()

            def put_row(r, c):
                d = xs_v[r, pl.ds(0, _LANES)].astype(jnp.float32)
                row_v[r, pl.ds(0, _LANES)] = d
                for f in range(_FIELD_NUM):
                    row_v[r, pl.ds(_SPARSE_START + f * _DIM, _DIM)] = (
                        emb_v[f, r, pl.ds(0, _DIM)]
                    )
                return c

            lax.fori_loop(0, _CH, put_row, 0)

            pltpu.sync_copy(row_v, out_hbm.at[pl.ds(base, _CH)])
            return carry

        lax.fori_loop(0, _NITER, chunk, 0)

    return k(x, tables)


def kernel(X, tables):
    return _sc_embed(X, tables)


# speedup vs baseline: 1.1627x; 1.0086x over previous
import functools

import jax
import jax.numpy as jnp
from jax import lax
from jax.experimental import pallas as pl
from jax.experimental.pallas import tpu as pltpu
from jax.experimental.pallas import tpu_sc as plsc

_B = 16384
_SPARSE_START = 13
_FIELD_NUM = 26
_VOCAB = 100000
_DIM = 16
_XCOLS = _SPARSE_START + _FIELD_NUM

_NC = 2
_NS = 16
_NW = _NC * _NS
_LANES = 16

_ROWS_PER_W = _B // _NW
_CH = 128
_NITER = _ROWS_PER_W // _CH

_OUT_COLS = _SPARSE_START + _FIELD_NUM * _DIM


def _sc_embed(x_flat, table_flat):
    mesh = plsc.VectorSubcoreMesh(core_axis_name="c", subcore_axis_name="s")

    @functools.partial(
        pl.kernel,
        mesh=mesh,
        compiler_params=pltpu.CompilerParams(
            use_tc_tiling_on_sc=False, needs_layout_passes=False
        ),
        out_type=jax.ShapeDtypeStruct((_B * _OUT_COLS,), jnp.float32),
        scratch_types=[
            pltpu.VMEM((_CH * _XCOLS,), jnp.int32),
            pltpu.VMEM((_FIELD_NUM, _CH), jnp.int32),
            pltpu.VMEM((_FIELD_NUM, _CH, _DIM), jnp.float32),
            pltpu.VMEM((_CH * _OUT_COLS,), jnp.float32),
            pltpu.SemaphoreType.DMA,
        ],
    )
    def k(x_hbm, tab_hbm, out_hbm, xs_v, idx_v, emb_v, row_v, gsem):
        wid = lax.axis_index("s") * _NC + lax.axis_index("c")
        w_base = wid * _ROWS_PER_W

        def chunk(it, carry):
            base = pl.multiple_of(w_base + it * _CH, _CH)

            pltpu.sync_copy(x_hbm.at[pl.ds(base * _XCOLS, _CH * _XCOLS)], xs_v)

            row_addr = lax.iota(jnp.int32, _LANES) * _XCOLS
            for f in range(_FIELD_NUM):
                bias = f * _VOCAB
                for j in range(_CH // _LANES):
                    addr = row_addr + (j * _LANES * _XCOLS + _SPARSE_START + f)
                    vals = plsc.load_gather(xs_v, [addr])
                    idx_v[f, pl.ds(j * _LANES, _LANES)] = vals + bias

            handles = []
            for f in range(_FIELD_NUM):
                handles.append(
                    pltpu.async_copy(tab_hbm.at[idx_v.at[f]], emb_v.at[f], gsem)
                )
            for h in handles:
                h.wait()

            def put_row(r, c):
                d = xs_v[pl.ds(r * _XCOLS, _LANES)].astype(jnp.float32)
                row_v[pl.ds(r * _OUT_COLS, _LANES)] = d
                for f in range(_FIELD_NUM):
                    row_v[pl.ds(r * _OUT_COLS + _SPARSE_START + f * _DIM, _DIM)] = (
                        emb_v[f, r, pl.ds(0, _DIM)]
                    )
                return c

            lax.fori_loop(0, _CH, put_row, 0)

            pltpu.sync_copy(
                row_v,
                out_hbm.at[pl.ds(pl.multiple_of(base * _OUT_COLS, 8), _CH * _OUT_COLS)],
            )
            return carry

        lax.fori_loop(0, _NITER, chunk, 0)

    return k(x_flat, table_flat)


def kernel(X, tables):
    x_flat = X.reshape(_B * _XCOLS)
    table_flat = tables.reshape(_FIELD_NUM * _VOCAB, _DIM)
    out_flat = _sc_embed(x_flat, table_flat)
    return out_flat.reshape(_B, _OUT_COLS)
